# baseline (speedup 1.0000x reference)
import jax
import jax.numpy as jnp
from jax import lax
from jax.experimental import pallas as pl
from jax.experimental.pallas import tpu as pltpu
from jax.experimental.pallas import tpu_sc as plsc

B = 16
H_KV = 2
MAX_LEN = 2048
HEAD_DIM = 128
NW = 32
ROWS = 4 * B * H_KV * MAX_LEN


CHUNK = 128
NBUF = 6
NCHUNK = 4 * MAX_LEN // CHUNK


def _body(c0, c1, c2, c3, rows_hbm, idx_hbm, out,
          b0, b1, b2, b3, b4, b5, idx_v, rows_v, sem_in, sem_out, sem_row):
    w = lax.axis_index("s") * 2 + lax.axis_index("c")
    caches = (c0, c1, c2, c3)
    bufs = (b0, b1, b2, b3, b4, b5)

    def src_slice(i):
        c, k = divmod(i, MAX_LEN // CHUNK)
        return caches[c].at[pl.ds(w * MAX_LEN + k * CHUNK, CHUNK)]

    def dst_slice(i):
        c, k = divmod(i, MAX_LEN // CHUNK)
        return out.at[pl.ds((c * NW + w) * MAX_LEN + k * CHUNK, CHUNK)]

    in_cp = [None] * NBUF
    out_cp = [None] * NBUF
    for i in range(NBUF):
        in_cp[i] = pltpu.async_copy(src_slice(i), bufs[i], sem_in.at[i])
    for i in range(NCHUNK):
        j = i % NBUF
        in_cp[j].wait()
        out_cp[j] = pltpu.async_copy(bufs[j], dst_slice(i), sem_out.at[j])
        nxt = i + NBUF
        if nxt < NCHUNK:
            out_cp[j].wait()
            in_cp[j] = pltpu.async_copy(src_slice(nxt), bufs[j], sem_in.at[j])
    for i in range(NCHUNK - NBUF, NCHUNK):
        out_cp[i % NBUF].wait()
    pltpu.sync_copy(idx_hbm.at[w], idx_v)
    pltpu.sync_copy(rows_hbm.at[w], rows_v)
    pltpu.async_copy(rows_v, out.at[idx_v], sem_row).wait()


def kernel(k_cache_0, v_cache_0, k_cache_1, v_cache_1,
           new_k_0, new_v_0, new_k_1, new_v_1,
           position_ids, seq_ids):
    del seq_ids
    pos = position_ids[:, 0].astype(jnp.int32)

    flat = lambda c: c.reshape(B * H_KV * MAX_LEN, HEAD_DIM)
    new_rows = jnp.stack(
        [new_k_0[:, :, 0], new_v_0[:, :, 0], new_k_1[:, :, 0], new_v_1[:, :, 0]],
        axis=2,
    ).reshape(B * H_KV, 4, HEAD_DIM)
    bh = jnp.arange(B * H_KV, dtype=jnp.int32)
    c = jnp.arange(4, dtype=jnp.int32)
    dest_idx = (c[None, :] * NW + bh[:, None]) * MAX_LEN + pos[bh // H_KV][:, None]

    mesh = plsc.VectorSubcoreMesh(core_axis_name="c", subcore_axis_name="s")
    out = pl.kernel(
        _body,
        out_type=jax.ShapeDtypeStruct((ROWS, HEAD_DIM), jnp.float32),
        mesh=mesh,
        scratch_types=[
            pltpu.VMEM((CHUNK, HEAD_DIM), jnp.float32),
            pltpu.VMEM((CHUNK, HEAD_DIM), jnp.float32),
            pltpu.VMEM((CHUNK, HEAD_DIM), jnp.float32),
            pltpu.VMEM((CHUNK, HEAD_DIM), jnp.float32),
            pltpu.VMEM((CHUNK, HEAD_DIM), jnp.float32),
            pltpu.VMEM((CHUNK, HEAD_DIM), jnp.float32),
            pltpu.VMEM((4,), jnp.int32),
            pltpu.VMEM((4, HEAD_DIM), jnp.float32),
            pltpu.SemaphoreType.DMA((NBUF,)),
            pltpu.SemaphoreType.DMA((NBUF,)),
            pltpu.SemaphoreType.DMA,
        ],
    )(flat(k_cache_0), flat(v_cache_0), flat(k_cache_1), flat(v_cache_1),
      new_rows, dest_idx)
    return out.reshape(4, B, H_KV, MAX_LEN, HEAD_DIM)


# speedup vs baseline: 1.0188x; 1.0188x over previous
import jax
import jax.numpy as jnp
from jax import lax
from jax.experimental import pallas as pl
from jax.experimental.pallas import tpu as pltpu
from jax.experimental.pallas import tpu_sc as plsc

B = 16
H_KV = 2
MAX_LEN = 2048
HEAD_DIM = 128
NW = 32
ROWS = 4 * B * H_KV * MAX_LEN


CHUNK = 256
NBUF = 3
NCHUNK = 4 * MAX_LEN // CHUNK


def _body(c0, c1, c2, c3, rows_hbm, idx_hbm, out,
          b0, b1, b2, idx_v, rows_v, sem_in, sem_out, sem_row):
    w = lax.axis_index("s") * 2 + lax.axis_index("c")
    caches = (c0, c1, c2, c3)
    bufs = (b0, b1, b2)

    def src_slice(i):
        c, k = divmod(i, MAX_LEN // CHUNK)
        return caches[c].at[pl.ds(w * MAX_LEN + k * CHUNK, CHUNK)]

    def dst_slice(i):
        c, k = divmod(i, MAX_LEN // CHUNK)
        return out.at[pl.ds((c * NW + w) * MAX_LEN + k * CHUNK, CHUNK)]

    in_cp = [None] * NBUF
    out_cp = [None] * NBUF
    for i in range(NBUF):
        in_cp[i] = pltpu.async_copy(src_slice(i), bufs[i], sem_in.at[i])
    for i in range(NCHUNK):
        j = i % NBUF
        in_cp[j].wait()
        out_cp[j] = pltpu.async_copy(bufs[j], dst_slice(i), sem_out.at[j])
        nxt = i + NBUF
        if nxt < NCHUNK:
            out_cp[j].wait()
            in_cp[j] = pltpu.async_copy(src_slice(nxt), bufs[j], sem_in.at[j])
    for i in range(NCHUNK - NBUF, NCHUNK):
        out_cp[i % NBUF].wait()
    pltpu.sync_copy(idx_hbm.at[w], idx_v)
    pltpu.sync_copy(rows_hbm.at[w], rows_v)
    pltpu.async_copy(rows_v, out.at[idx_v], sem_row).wait()


def kernel(k_cache_0, v_cache_0, k_cache_1, v_cache_1,
           new_k_0, new_v_0, new_k_1, new_v_1,
           position_ids, seq_ids):
    del seq_ids
    pos = position_ids[:, 0].astype(jnp.int32)

    flat = lambda c: c.reshape(B * H_KV * MAX_LEN, HEAD_DIM)
    new_rows = jnp.stack(
        [new_k_0[:, :, 0], new_v_0[:, :, 0], new_k_1[:, :, 0], new_v_1[:, :, 0]],
        axis=2,
    ).reshape(B * H_KV, 4, HEAD_DIM)
    bh = jnp.arange(B * H_KV, dtype=jnp.int32)
    c = jnp.arange(4, dtype=jnp.int32)
    dest_idx = (c[None, :] * NW + bh[:, None]) * MAX_LEN + pos[bh // H_KV][:, None]

    mesh = plsc.VectorSubcoreMesh(core_axis_name="c", subcore_axis_name="s")
    out = pl.kernel(
        _body,
        out_type=jax.ShapeDtypeStruct((ROWS, HEAD_DIM), jnp.float32),
        mesh=mesh,
        scratch_types=[
            pltpu.VMEM((CHUNK, HEAD_DIM), jnp.float32),
            pltpu.VMEM((CHUNK, HEAD_DIM), jnp.float32),
            pltpu.VMEM((CHUNK, HEAD_DIM), jnp.float32),
            pltpu.VMEM((4,), jnp.int32),
            pltpu.VMEM((4, HEAD_DIM), jnp.float32),
            pltpu.SemaphoreType.DMA((NBUF,)),
            pltpu.SemaphoreType.DMA((NBUF,)),
            pltpu.SemaphoreType.DMA,
        ],
    )(flat(k_cache_0), flat(v_cache_0), flat(k_cache_1), flat(v_cache_1),
      new_rows, dest_idx)
    return out.reshape(4, B, H_KV, MAX_LEN, HEAD_DIM)


# speedup vs baseline: 1.0736x; 1.0538x over previous
import jax
import jax.numpy as jnp
from jax import lax
from jax.experimental import pallas as pl
from jax.experimental.pallas import tpu as pltpu
from jax.experimental.pallas import tpu_sc as plsc

B = 16
H_KV = 2
MAX_LEN = 2048
HEAD_DIM = 128
NW = 32
ROWS = 4 * B * H_KV * MAX_LEN


CHUNK = 256
NBUF = 3
NCHUNK = 4 * MAX_LEN // CHUNK


def _body(c0, c1, c2, c3, rows_hbm, idx_hbm, out,
          shared, idx_v, rows_v, sem_in, sem_out, sem_row):
    s = lax.axis_index("s")
    w = s * 2 + lax.axis_index("c")
    caches = (c0, c1, c2, c3)
    bufs = tuple(shared.at[s, j] for j in range(NBUF))

    def src_slice(i):
        c, k = divmod(i, MAX_LEN // CHUNK)
        return caches[c].at[pl.ds(w * MAX_LEN + k * CHUNK, CHUNK)]

    def dst_slice(i):
        c, k = divmod(i, MAX_LEN // CHUNK)
        return out.at[pl.ds((c * NW + w) * MAX_LEN + k * CHUNK, CHUNK)]

    in_cp = [None] * NBUF
    out_cp = [None] * NBUF
    for i in range(NBUF):
        in_cp[i] = pltpu.async_copy(src_slice(i), bufs[i], sem_in.at[i])
    for i in range(NCHUNK):
        j = i % NBUF
        in_cp[j].wait()
        out_cp[j] = pltpu.async_copy(bufs[j], dst_slice(i), sem_out.at[j])
        nxt = i + NBUF
        if nxt < NCHUNK:
            out_cp[j].wait()
            in_cp[j] = pltpu.async_copy(src_slice(nxt), bufs[j], sem_in.at[j])
    for i in range(NCHUNK - NBUF, NCHUNK):
        out_cp[i % NBUF].wait()
    pltpu.sync_copy(idx_hbm.at[w], idx_v)
    pltpu.sync_copy(rows_hbm.at[w], rows_v)
    pltpu.async_copy(rows_v, out.at[idx_v], sem_row).wait()


def kernel(k_cache_0, v_cache_0, k_cache_1, v_cache_1,
           new_k_0, new_v_0, new_k_1, new_v_1,
           position_ids, seq_ids):
    del seq_ids
    pos = position_ids[:, 0].astype(jnp.int32)

    flat = lambda c: c.reshape(B * H_KV * MAX_LEN, HEAD_DIM)
    new_rows = jnp.stack(
        [new_k_0[:, :, 0], new_v_0[:, :, 0], new_k_1[:, :, 0], new_v_1[:, :, 0]],
        axis=2,
    ).reshape(B * H_KV, 4, HEAD_DIM)
    bh = jnp.arange(B * H_KV, dtype=jnp.int32)
    c = jnp.arange(4, dtype=jnp.int32)
    dest_idx = (c[None, :] * NW + bh[:, None]) * MAX_LEN + pos[bh // H_KV][:, None]

    mesh = plsc.VectorSubcoreMesh(core_axis_name="c", subcore_axis_name="s")
    out = pl.kernel(
        _body,
        out_type=jax.ShapeDtypeStruct((ROWS, HEAD_DIM), jnp.float32),
        mesh=mesh,
        scratch_types=[
            pltpu.VMEM_SHARED((16, NBUF, CHUNK, HEAD_DIM), jnp.float32),
            pltpu.VMEM((4,), jnp.int32),
            pltpu.VMEM((4, HEAD_DIM), jnp.float32),
            pltpu.SemaphoreType.DMA((NBUF,)),
            pltpu.SemaphoreType.DMA((NBUF,)),
            pltpu.SemaphoreType.DMA,
        ],
    )(flat(k_cache_0), flat(v_cache_0), flat(k_cache_1), flat(v_cache_1),
      new_rows, dest_idx)
    return out.reshape(4, B, H_KV, MAX_LEN, HEAD_DIM)
